# baseline (speedup 1.0000x reference)
import functools

import jax
import jax.numpy as jnp
from jax import lax
from jax.experimental import pallas as pl
from jax.experimental.pallas import tpu as pltpu
from jax.experimental.pallas import tpu_sc as plsc

GAUSSIAN_STD = 0.3
EPS = 1e-05
NB = 2
HT = 8


def _body(img_ref, apc_ref, shp_ref, zeta_ref, idx_ref, sco_ref, mref, dref,
          supp_ref, *, K, C):
    img = img_ref[...]
    x = apc_ref[...]
    m0 = shp_ref[...]
    z = zeta_ref[...][None]
    z = jnp.transpose(z, (2, 1, 0, 3))

    H, W = m0.shape[-2], m0.shape[-1]
    WP = mref.shape[-1]
    ginv = 1.0 / (GAUSSIAN_STD * GAUSSIAN_STD)

    dc = x[:, :, 0] - img[None, :, 0]
    d = dc * dc
    for c in range(1, C):
        dc = x[:, :, c] - img[None, :, c]
        d = d + dc * dc

    pad = jnp.zeros((K, NB, H, WP - W), jnp.float32)
    mref[...] = jnp.concatenate([m0, pad], axis=-1)
    dref[...] = jnp.concatenate([d, pad], axis=-1)

    kio = jax.lax.broadcasted_iota(jnp.int32, (K, NB, 1, 1), 0)
    coefs = jnp.ones((K, NB, 1, 1), dtype=jnp.float32)
    idx_out = jnp.zeros((NB, K, 1), dtype=jnp.int32)
    tio = jax.lax.broadcasted_iota(jnp.int32, (NB, K, 1), 1)

    have_supp = False
    for t in range(K):
        vds_l, va_l, vm_l = [], [], []
        for k in range(K):
            a_s = a_a = a_m = None
            for h in range(0, H, HT):
                mt = mref[k, :, h:h + HT, :]
                if have_supp:
                    mt = mt * supp_ref[:, h:h + HT, :]
                    if t + 1 < K:
                        mref[k, :, h:h + HT, :] = mt
                ms = mt * dref[k, :, h:h + HT, :]
                if a_s is None:
                    a_s, a_a, a_m = ms, mt, mt
                else:
                    a_s = a_s + ms
                    a_a = a_a + mt
                    a_m = jnp.maximum(a_m, mt)
            vds_l.append(jnp.sum(a_s, axis=(1, 2), keepdims=True))
            va_l.append(jnp.sum(a_a, axis=(1, 2), keepdims=True))
            vm_l.append(jnp.max(a_m, axis=(1, 2), keepdims=True))
        vds = jnp.stack(vds_l)
        va = jnp.stack(va_l)
        vm = jnp.stack(vm_l)

        s = coefs * vm * z * jnp.exp(-0.5 * ginv * vds / (va + EPS))

        mx = jnp.max(s, axis=0, keepdims=True)
        idx = jnp.min(jnp.where(s == mx, kio, K), axis=0, keepdims=True)

        idx_out = jnp.where(tio == t, idx[0], idx_out)
        sco_ref[:, t * K:(t + 1) * K, :] = jnp.transpose(s[:, :, :, 0], (1, 0, 2))

        if t + 1 < K:
            rows = [mref[idx[0, b, 0, 0], b] for b in range(NB)]
            supp_ref[...] = 1.0 - jnp.stack(rows, axis=0)
            have_supp = True
        coefs = jnp.where(kio == idx, -1.0, coefs)

    idx_ref[...] = idx_out


def _sc_probe(apc):
    K, B, C, H, W = apc.shape
    mesh = plsc.VectorSubcoreMesh(core_axis_name="c", subcore_axis_name="s")

    @functools.partial(
        pl.kernel,
        mesh=mesh,
        out_type=jax.ShapeDtypeStruct((32, 16), jnp.float32),
        scratch_types=[
            pltpu.VMEM((C, H, W), jnp.float32),
            pltpu.VMEM((16,), jnp.float32),
        ],
    )
    def body(apc_hbm, out_hbm, buf, accv):
        wid = lax.axis_index("s") * 2 + lax.axis_index("c")
        k = wid // 4
        bg = wid % 4
        acc = jnp.zeros((16,), jnp.float32)
        for j in range(16):
            pltpu.sync_copy(apc_hbm.at[k, bg * 16 + j], buf)
            acc = acc + buf[0, 0, pl.ds(0, 16)]
        accv[...] = acc
        pltpu.sync_copy(accv, out_hbm.at[wid])

    return body(apc)


@jax.jit
def kernel(images, apc, shp, zeta):
    K, B, C, H, W = apc.shape
    WP = (W + 127) // 128 * 128
    sc_tok = _sc_probe(apc)

    shp4 = shp.reshape(K, B, H, W)
    zeta2 = zeta.transpose(1, 0, 2)

    out_idx, out_sco = pl.pallas_call(
        functools.partial(_body, K=K, C=C),
        grid=(B // NB,),
        in_specs=[
            pl.BlockSpec((NB, C, H, W), lambda b: (b, 0, 0, 0)),
            pl.BlockSpec((K, NB, C, H, W), lambda b: (0, b, 0, 0, 0)),
            pl.BlockSpec((K, NB, H, W), lambda b: (0, b, 0, 0)),
            pl.BlockSpec((NB, K, 1), lambda b: (b, 0, 0)),
        ],
        out_specs=[
            pl.BlockSpec((NB, K, 1), lambda b: (b, 0, 0)),
            pl.BlockSpec((NB, K * K, 1), lambda b: (b, 0, 0)),
        ],
        out_shape=[
            jax.ShapeDtypeStruct((B, K, 1), jnp.int32),
            jax.ShapeDtypeStruct((B, K * K, 1), jnp.float32),
        ],
        scratch_shapes=[
            pltpu.VMEM((K, NB, H, WP), jnp.float32),
            pltpu.VMEM((K, NB, H, WP), jnp.float32),
            pltpu.VMEM((NB, H, WP), jnp.float32),
        ],
    )(images, apc, shp4, zeta2)

    indices_all = out_idx.transpose(1, 0, 2)
    scores_all = out_sco.transpose(1, 0, 2).reshape(K, K, B, 1)
    scores_all = scores_all + jnp.sum(sc_tok) * 1e-30
    return indices_all, scores_all


# speedup vs baseline: 1.2327x; 1.2327x over previous
import functools

import jax
import jax.numpy as jnp
from jax.experimental import pallas as pl
from jax.experimental.pallas import tpu as pltpu

GAUSSIAN_STD = 0.3
EPS = 1e-05
NB = 2
HT = 16


def _body(img_ref, apc_ref, shp_ref, zeta_ref, idx_ref, sco_ref, mref, dref,
          supp_ref, *, K, C):
    img = img_ref[...]
    x = apc_ref[...]
    m0 = shp_ref[...]
    z = zeta_ref[...][None]
    z = jnp.transpose(z, (2, 1, 0, 3))

    H, W = m0.shape[-2], m0.shape[-1]
    WP = mref.shape[-1]
    ginv = 1.0 / (GAUSSIAN_STD * GAUSSIAN_STD)

    dc = x[:, :, 0] - img[None, :, 0]
    d = dc * dc
    for c in range(1, C):
        dc = x[:, :, c] - img[None, :, c]
        d = d + dc * dc

    pad = jnp.zeros((K, NB, H, WP - W), jnp.float32)
    mref[...] = jnp.concatenate([m0, pad], axis=-1)
    dref[...] = jnp.concatenate([d, pad], axis=-1)

    kio = jax.lax.broadcasted_iota(jnp.int32, (K, NB, 1, 1), 0)
    coefs = jnp.ones((K, NB, 1, 1), dtype=jnp.float32)
    idx_out = jnp.zeros((NB, K, 1), dtype=jnp.int32)
    tio = jax.lax.broadcasted_iota(jnp.int32, (NB, K, 1), 1)

    have_supp = False
    for t in range(K):
        vds_l, va_l, vm_l = [], [], []
        for k in range(K):
            a_s = a_a = a_m = None
            for h in range(0, H, HT):
                mt = mref[k, :, h:h + HT, :]
                if have_supp:
                    mt = mt * supp_ref[:, h:h + HT, :]
                    if t + 1 < K:
                        mref[k, :, h:h + HT, :] = mt
                ms = mt * dref[k, :, h:h + HT, :]
                if a_s is None:
                    a_s, a_a, a_m = ms, mt, mt
                else:
                    a_s = a_s + ms
                    a_a = a_a + mt
                    a_m = jnp.maximum(a_m, mt)
            vds_l.append(jnp.sum(a_s, axis=(1, 2), keepdims=True))
            va_l.append(jnp.sum(a_a, axis=(1, 2), keepdims=True))
            vm_l.append(jnp.max(a_m, axis=(1, 2), keepdims=True))
        vds = jnp.stack(vds_l)
        va = jnp.stack(va_l)
        vm = jnp.stack(vm_l)

        s = coefs * vm * z * jnp.exp(-0.5 * ginv * vds / (va + EPS))

        mx = jnp.max(s, axis=0, keepdims=True)
        idx = jnp.min(jnp.where(s == mx, kio, K), axis=0, keepdims=True)

        idx_out = jnp.where(tio == t, idx[0], idx_out)
        sco_ref[:, t * K:(t + 1) * K, :] = jnp.transpose(s[:, :, :, 0], (1, 0, 2))

        if t + 1 < K:
            rows = [mref[idx[0, b, 0, 0], b] for b in range(NB)]
            supp_ref[...] = 1.0 - jnp.stack(rows, axis=0)
            have_supp = True
        coefs = jnp.where(kio == idx, -1.0, coefs)

    idx_ref[...] = idx_out


@jax.jit
def kernel(images, apc, shp, zeta):
    K, B, C, H, W = apc.shape
    WP = (W + 127) // 128 * 128

    shp4 = shp.reshape(K, B, H, W)
    zeta2 = zeta.transpose(1, 0, 2)

    out_idx, out_sco = pl.pallas_call(
        functools.partial(_body, K=K, C=C),
        grid=(B // NB,),
        in_specs=[
            pl.BlockSpec((NB, C, H, W), lambda b: (b, 0, 0, 0)),
            pl.BlockSpec((K, NB, C, H, W), lambda b: (0, b, 0, 0, 0)),
            pl.BlockSpec((K, NB, H, W), lambda b: (0, b, 0, 0)),
            pl.BlockSpec((NB, K, 1), lambda b: (b, 0, 0)),
        ],
        out_specs=[
            pl.BlockSpec((NB, K, 1), lambda b: (b, 0, 0)),
            pl.BlockSpec((NB, K * K, 1), lambda b: (b, 0, 0)),
        ],
        out_shape=[
            jax.ShapeDtypeStruct((B, K, 1), jnp.int32),
            jax.ShapeDtypeStruct((B, K * K, 1), jnp.float32),
        ],
        scratch_shapes=[
            pltpu.VMEM((K, NB, H, WP), jnp.float32),
            pltpu.VMEM((K, NB, H, WP), jnp.float32),
            pltpu.VMEM((NB, H, WP), jnp.float32),
        ],
    )(images, apc, shp4, zeta2)

    indices_all = out_idx.transpose(1, 0, 2)
    scores_all = out_sco.transpose(1, 0, 2).reshape(K, K, B, 1)
    return indices_all, scores_all
